# initial kernel scaffold (unmeasured)
import jax
import jax.numpy as jnp
from jax import lax
import jax.experimental.pallas as pl
from jax.experimental.pallas import tpu as pltpu

N_DEV = 16


def _ag_body(x_ref, w_ref, gx_ref, gw_ref, send_x, recv_x, send_w, recv_w):
    my = lax.axis_index("i")
    left = lax.rem(my + N_DEV - 1, N_DEV)
    right = lax.rem(my + 1, N_DEV)

    barrier = pltpu.get_barrier_semaphore()
    pl.semaphore_signal(barrier, inc=1, device_id=(left,),
                        device_id_type=pl.DeviceIdType.MESH)
    pl.semaphore_signal(barrier, inc=1, device_id=(right,),
                        device_id_type=pl.DeviceIdType.MESH)
    pl.semaphore_wait(barrier, 2)

    gx_ref[my] = x_ref[...]
    gw_ref[my] = w_ref[...]

    for h in range(N_DEV - 1):
        src = lax.rem(my - h + N_DEV, N_DEV)
        rx = pltpu.make_async_remote_copy(
            src_ref=gx_ref.at[src], dst_ref=gx_ref.at[src],
            send_sem=send_x.at[h], recv_sem=recv_x.at[h],
            device_id=(right,), device_id_type=pl.DeviceIdType.MESH)
        rw = pltpu.make_async_remote_copy(
            src_ref=gw_ref.at[src], dst_ref=gw_ref.at[src],
            send_sem=send_w.at[h], recv_sem=recv_w.at[h],
            device_id=(right,), device_id_type=pl.DeviceIdType.MESH)
        rx.start()
        rw.start()
        rx.wait()
        rw.wait()


def _allgather_inputs(x, w_mat):
    m, k_per = x.shape
    _, n = w_mat.shape
    return pl.pallas_call(
        _ag_body,
        out_shape=[
            jax.ShapeDtypeStruct((N_DEV, m, k_per), jnp.int8),
            jax.ShapeDtypeStruct((N_DEV, k_per, n), jnp.int8),
        ],
        in_specs=[
            pl.BlockSpec(memory_space=pltpu.VMEM),
            pl.BlockSpec(memory_space=pltpu.VMEM),
        ],
        out_specs=[
            pl.BlockSpec(memory_space=pltpu.VMEM),
            pl.BlockSpec(memory_space=pltpu.VMEM),
        ],
        scratch_shapes=[
            pltpu.SemaphoreType.DMA((N_DEV - 1,)),
            pltpu.SemaphoreType.DMA((N_DEV - 1,)),
            pltpu.SemaphoreType.DMA((N_DEV - 1,)),
            pltpu.SemaphoreType.DMA((N_DEV - 1,)),
        ],
        compiler_params=pltpu.CompilerParams(collective_id=0),
    )(x, w_mat)


def _epi_body(s_ref, acc_ref, out_ref):
    y = acc_ref[...] * s_ref[0, 0]
    out_ref[...] = y * jax.nn.sigmoid(y)


def _epilogue(acc, scale):
    m, n = acc.shape
    blocks = 16
    bm = m // blocks
    return pl.pallas_call(
        _epi_body,
        out_shape=jax.ShapeDtypeStruct((m, n), jnp.float32),
        grid=(blocks,),
        in_specs=[
            pl.BlockSpec((1, 1), lambda i: (0, 0)),
            pl.BlockSpec((bm, n), lambda i: (i, 0)),
        ],
        out_specs=pl.BlockSpec((bm, n), lambda i: (i, 0)),
    )(scale, acc)


def kernel(x, w_mat, scale_x, scale_w):
    gx, gw = _allgather_inputs(x, w_mat)
    acc = lax.dot_general(
        gx.astype(jnp.bfloat16), gw.astype(jnp.bfloat16),
        dimension_numbers=(((0, 2), (0, 1)), ((), ())),
        preferred_element_type=jnp.float32,
    )
    scale = jnp.reshape(scale_x * scale_w, (1, 1))
    return _epilogue(acc, scale)


# baseline (device time: 1048359 ns/iter reference)
import jax
import jax.numpy as jnp
from jax import lax
import jax.experimental.pallas as pl
from jax.experimental.pallas import tpu as pltpu

N_DEV = 16


def _ag_body(x_ref, w_ref, gx_ref, gw_ref, send_x, recv_x, send_w, recv_w):
    my = lax.axis_index("i")
    left = lax.rem(my + N_DEV - 1, N_DEV)
    right = lax.rem(my + 1, N_DEV)

    barrier = pltpu.get_barrier_semaphore()
    pl.semaphore_signal(barrier, inc=1, device_id=(left,),
                        device_id_type=pl.DeviceIdType.MESH)
    pl.semaphore_signal(barrier, inc=1, device_id=(right,),
                        device_id_type=pl.DeviceIdType.MESH)
    pl.semaphore_wait(barrier, 2)

    gx_ref[my] = x_ref[...]
    gw_ref[my] = w_ref[...]

    for h in range(N_DEV - 1):
        src = lax.rem(my - h + N_DEV, N_DEV)
        rx = pltpu.make_async_remote_copy(
            src_ref=gx_ref.at[src], dst_ref=gx_ref.at[src],
            send_sem=send_x.at[h], recv_sem=recv_x.at[h],
            device_id=(right,), device_id_type=pl.DeviceIdType.MESH)
        rw = pltpu.make_async_remote_copy(
            src_ref=gw_ref.at[src], dst_ref=gw_ref.at[src],
            send_sem=send_w.at[h], recv_sem=recv_w.at[h],
            device_id=(right,), device_id_type=pl.DeviceIdType.MESH)
        rx.start()
        rw.start()
        rx.wait()
        rw.wait()


def _allgather_inputs(x, w_mat):
    m, k_per = x.shape
    _, n = w_mat.shape
    return pl.pallas_call(
        _ag_body,
        out_shape=[
            jax.ShapeDtypeStruct((N_DEV, m, k_per), jnp.int8),
            jax.ShapeDtypeStruct((N_DEV, k_per, n), jnp.int8),
        ],
        in_specs=[
            pl.BlockSpec(memory_space=pltpu.VMEM),
            pl.BlockSpec(memory_space=pltpu.VMEM),
        ],
        out_specs=[
            pl.BlockSpec(memory_space=pltpu.VMEM),
            pl.BlockSpec(memory_space=pltpu.VMEM),
        ],
        scratch_shapes=[
            pltpu.SemaphoreType.DMA((N_DEV - 1,)),
            pltpu.SemaphoreType.DMA((N_DEV - 1,)),
            pltpu.SemaphoreType.DMA((N_DEV - 1,)),
            pltpu.SemaphoreType.DMA((N_DEV - 1,)),
        ],
        compiler_params=pltpu.CompilerParams(collective_id=0),
    )(x, w_mat)


def _epi_body(s_ref, acc_ref, out_ref):
    y = acc_ref[...] * s_ref[0, 0]
    out_ref[...] = y * jax.nn.sigmoid(y)


def _epilogue(acc, scale):
    m, n = acc.shape
    blocks = 32
    bm = m // blocks
    return pl.pallas_call(
        _epi_body,
        out_shape=jax.ShapeDtypeStruct((m, n), jnp.float32),
        grid=(blocks,),
        in_specs=[
            pl.BlockSpec((1, 1), lambda i: (0, 0)),
            pl.BlockSpec((bm, n), lambda i: (i, 0)),
        ],
        out_specs=pl.BlockSpec((bm, n), lambda i: (i, 0)),
    )(scale, acc)


def kernel(x, w_mat, scale_x, scale_w):
    gx, gw = _allgather_inputs(x, w_mat)
    acc = lax.dot_general(
        gx.astype(jnp.bfloat16), gw.astype(jnp.bfloat16),
        dimension_numbers=(((0, 2), (0, 1)), ((), ())),
        preferred_element_type=jnp.float32,
    )
    scale = jnp.reshape(scale_x * scale_w, (1, 1))
    return _epilogue(acc, scale)


# device time: 723278 ns/iter; 1.4495x vs baseline; 1.4495x over previous
import jax
import jax.numpy as jnp
from jax import lax
import jax.experimental.pallas as pl
from jax.experimental.pallas import tpu as pltpu

N_DEV = 16


def _ag_body(x_ref, w_ref, gx_ref, gw_ref,
             sx_r, rx_r, sx_l, rx_l, sw_r, rw_r, sw_l, rw_l):
    my = lax.axis_index("i")
    left = lax.rem(my + N_DEV - 1, N_DEV)
    right = lax.rem(my + 1, N_DEV)

    barrier = pltpu.get_barrier_semaphore()
    pl.semaphore_signal(barrier, inc=1, device_id=(left,),
                        device_id_type=pl.DeviceIdType.MESH)
    pl.semaphore_signal(barrier, inc=1, device_id=(right,),
                        device_id_type=pl.DeviceIdType.MESH)
    pl.semaphore_wait(barrier, 2)

    m, k_per = x_ref.shape
    _, n = w_ref.shape
    mh = m // 2
    nh = n // 2

    gx_ref[:, pl.ds(my * k_per, k_per)] = x_ref[...]
    gw_ref[pl.ds(my * k_per, k_per), :] = w_ref[...]

    for h in range(N_DEV - 1):
        src_r = lax.rem(my - h + N_DEV, N_DEV)
        src_l = lax.rem(my + h, N_DEV)
        cr = pl.ds(src_r * k_per, k_per)
        cl = pl.ds(src_l * k_per, k_per)
        rdmas = [
            pltpu.make_async_remote_copy(
                src_ref=gx_ref.at[pl.ds(0, mh), cr],
                dst_ref=gx_ref.at[pl.ds(0, mh), cr],
                send_sem=sx_r.at[h], recv_sem=rx_r.at[h],
                device_id=(right,), device_id_type=pl.DeviceIdType.MESH),
            pltpu.make_async_remote_copy(
                src_ref=gx_ref.at[pl.ds(mh, mh), cl],
                dst_ref=gx_ref.at[pl.ds(mh, mh), cl],
                send_sem=sx_l.at[h], recv_sem=rx_l.at[h],
                device_id=(left,), device_id_type=pl.DeviceIdType.MESH),
            pltpu.make_async_remote_copy(
                src_ref=gw_ref.at[cr, pl.ds(0, nh)],
                dst_ref=gw_ref.at[cr, pl.ds(0, nh)],
                send_sem=sw_r.at[h], recv_sem=rw_r.at[h],
                device_id=(right,), device_id_type=pl.DeviceIdType.MESH),
            pltpu.make_async_remote_copy(
                src_ref=gw_ref.at[cl, pl.ds(nh, nh)],
                dst_ref=gw_ref.at[cl, pl.ds(nh, nh)],
                send_sem=sw_l.at[h], recv_sem=rw_l.at[h],
                device_id=(left,), device_id_type=pl.DeviceIdType.MESH),
        ]
        for r in rdmas:
            r.start()
        for r in rdmas:
            r.wait()


def _allgather_inputs(x, w_mat):
    m, k_per = x.shape
    _, n = w_mat.shape
    return pl.pallas_call(
        _ag_body,
        out_shape=[
            jax.ShapeDtypeStruct((m, N_DEV * k_per), jnp.int8),
            jax.ShapeDtypeStruct((N_DEV * k_per, n), jnp.int8),
        ],
        in_specs=[
            pl.BlockSpec(memory_space=pltpu.VMEM),
            pl.BlockSpec(memory_space=pltpu.VMEM),
        ],
        out_specs=[
            pl.BlockSpec(memory_space=pltpu.VMEM),
            pl.BlockSpec(memory_space=pltpu.VMEM),
        ],
        scratch_shapes=[pltpu.SemaphoreType.DMA((N_DEV - 1,))] * 8,
        compiler_params=pltpu.CompilerParams(collective_id=0),
    )(x, w_mat)


def _epi_body(s_ref, acc_ref, out_ref):
    y = acc_ref[...] * s_ref[0, 0]
    out_ref[...] = y * jax.nn.sigmoid(y)


def _epilogue(acc, scale):
    m, n = acc.shape
    blocks = 32
    bm = m // blocks
    return pl.pallas_call(
        _epi_body,
        out_shape=jax.ShapeDtypeStruct((m, n), jnp.float32),
        grid=(blocks,),
        in_specs=[
            pl.BlockSpec((1, 1), lambda i: (0, 0)),
            pl.BlockSpec((bm, n), lambda i: (i, 0)),
        ],
        out_specs=pl.BlockSpec((bm, n), lambda i: (i, 0)),
    )(scale, acc)


def kernel(x, w_mat, scale_x, scale_w):
    gx, gw = _allgather_inputs(x, w_mat)
    acc = lax.dot_general(
        gx.astype(jnp.bfloat16), gw.astype(jnp.bfloat16),
        dimension_numbers=(((1,), (0,)), ((), ())),
        preferred_element_type=jnp.float32,
    )
    scale = jnp.reshape(scale_x * scale_w, (1, 1))
    return _epilogue(acc, scale)


# device time: 708850 ns/iter; 1.4790x vs baseline; 1.0204x over previous
import jax
import jax.numpy as jnp
from jax import lax
import jax.experimental.pallas as pl
from jax.experimental.pallas import tpu as pltpu

N_DEV = 16


def _ag_body(x_ref, w_ref, gx_ref, gw_ref,
             sx_r, rx_r, sx_l, rx_l, sw_r, rw_r, sw_l, rw_l):
    my = lax.axis_index("i")
    left = lax.rem(my + N_DEV - 1, N_DEV)
    right = lax.rem(my + 1, N_DEV)

    barrier = pltpu.get_barrier_semaphore()
    pl.semaphore_signal(barrier, inc=1, device_id=(left,),
                        device_id_type=pl.DeviceIdType.MESH)
    pl.semaphore_signal(barrier, inc=1, device_id=(right,),
                        device_id_type=pl.DeviceIdType.MESH)
    pl.semaphore_wait(barrier, 2)

    m, k_per = x_ref.shape
    _, n = w_ref.shape
    mh = m // 2
    nh = n // 2

    gx_ref[:, pl.ds(my * k_per, k_per)] = x_ref[...]
    gw_ref[pl.ds(my * k_per, k_per), :] = w_ref[...]

    all_rdmas = []
    for h in range(N_DEV - 1):
        src_r = lax.rem(my - h + N_DEV, N_DEV)
        src_l = lax.rem(my + h, N_DEV)
        cr = pl.ds(src_r * k_per, k_per)
        cl = pl.ds(src_l * k_per, k_per)
        rdmas = [
            pltpu.make_async_remote_copy(
                src_ref=gx_ref.at[pl.ds(0, mh), cr],
                dst_ref=gx_ref.at[pl.ds(0, mh), cr],
                send_sem=sx_r.at[h], recv_sem=rx_r.at[h],
                device_id=(right,), device_id_type=pl.DeviceIdType.MESH),
            pltpu.make_async_remote_copy(
                src_ref=gx_ref.at[pl.ds(mh, mh), cl],
                dst_ref=gx_ref.at[pl.ds(mh, mh), cl],
                send_sem=sx_l.at[h], recv_sem=rx_l.at[h],
                device_id=(left,), device_id_type=pl.DeviceIdType.MESH),
            pltpu.make_async_remote_copy(
                src_ref=gw_ref.at[cr, pl.ds(0, nh)],
                dst_ref=gw_ref.at[cr, pl.ds(0, nh)],
                send_sem=sw_r.at[h], recv_sem=rw_r.at[h],
                device_id=(right,), device_id_type=pl.DeviceIdType.MESH),
            pltpu.make_async_remote_copy(
                src_ref=gw_ref.at[cl, pl.ds(nh, nh)],
                dst_ref=gw_ref.at[cl, pl.ds(nh, nh)],
                send_sem=sw_l.at[h], recv_sem=rw_l.at[h],
                device_id=(left,), device_id_type=pl.DeviceIdType.MESH),
        ]
        for r in rdmas:
            r.start()
        for r in rdmas:
            r.wait_recv()
        all_rdmas.extend(rdmas)

    for r in all_rdmas:
        r.wait_send()


def _allgather_inputs(x, w_mat):
    m, k_per = x.shape
    _, n = w_mat.shape
    return pl.pallas_call(
        _ag_body,
        out_shape=[
            jax.ShapeDtypeStruct((m, N_DEV * k_per), jnp.int8),
            jax.ShapeDtypeStruct((N_DEV * k_per, n), jnp.int8),
        ],
        in_specs=[
            pl.BlockSpec(memory_space=pltpu.VMEM),
            pl.BlockSpec(memory_space=pltpu.VMEM),
        ],
        out_specs=[
            pl.BlockSpec(memory_space=pltpu.VMEM),
            pl.BlockSpec(memory_space=pltpu.VMEM),
        ],
        scratch_shapes=[pltpu.SemaphoreType.DMA((N_DEV - 1,))] * 8,
        compiler_params=pltpu.CompilerParams(collective_id=0),
    )(x, w_mat)


def _epi_body(s_ref, acc_ref, out_ref):
    y = acc_ref[...].astype(jnp.float32) * s_ref[0, 0]
    out_ref[...] = y * jax.nn.sigmoid(y)


def _epilogue(acc, scale):
    m, n = acc.shape
    blocks = 32
    bm = m // blocks
    return pl.pallas_call(
        _epi_body,
        out_shape=jax.ShapeDtypeStruct((m, n), jnp.float32),
        grid=(blocks,),
        in_specs=[
            pl.BlockSpec((1, 1), lambda i: (0, 0)),
            pl.BlockSpec((bm, n), lambda i: (i, 0)),
        ],
        out_specs=pl.BlockSpec((bm, n), lambda i: (i, 0)),
    )(scale, acc)


def kernel(x, w_mat, scale_x, scale_w):
    gx, gw = _allgather_inputs(x, w_mat)
    acc = lax.dot_general(
        gx.astype(jnp.bfloat16), gw.astype(jnp.bfloat16),
        dimension_numbers=(((1,), (0,)), ((), ())),
        preferred_element_type=jnp.float32,
    ).astype(jnp.bfloat16)
    scale = jnp.reshape(scale_x * scale_w, (1, 1))
    return _epilogue(acc, scale)


# device time: 689771 ns/iter; 1.5199x vs baseline; 1.0277x over previous
import jax
import jax.numpy as jnp
from jax import lax
import jax.experimental.pallas as pl
from jax.experimental.pallas import tpu as pltpu

N_DEV = 16


def _ag_body(x_ref, w_ref, gx_ref, gw_ref,
             sx_r, rx_r, sx_l, rx_l, sw_r, rw_r, sw_l, rw_l):
    my = lax.axis_index("i")
    left = lax.rem(my + N_DEV - 1, N_DEV)
    right = lax.rem(my + 1, N_DEV)

    barrier = pltpu.get_barrier_semaphore()
    pl.semaphore_signal(barrier, inc=1, device_id=(left,),
                        device_id_type=pl.DeviceIdType.MESH)
    pl.semaphore_signal(barrier, inc=1, device_id=(right,),
                        device_id_type=pl.DeviceIdType.MESH)
    pl.semaphore_wait(barrier, 2)

    m, k_per = x_ref.shape
    _, n = w_ref.shape
    mh = m // 2
    nh = n // 2

    gx_ref[:, pl.ds(my * k_per, k_per)] = x_ref[...]
    gw_ref[pl.ds(my * k_per, k_per), :] = w_ref[...]

    all_rdmas = []
    for h in range(N_DEV - 1):
        src_r = lax.rem(my - h + N_DEV, N_DEV)
        src_l = lax.rem(my + h, N_DEV)
        cr = pl.ds(src_r * k_per, k_per)
        cl = pl.ds(src_l * k_per, k_per)
        rdmas = [
            pltpu.make_async_remote_copy(
                src_ref=gx_ref.at[pl.ds(0, mh), cr],
                dst_ref=gx_ref.at[pl.ds(0, mh), cr],
                send_sem=sx_r.at[h], recv_sem=rx_r.at[h],
                device_id=(right,), device_id_type=pl.DeviceIdType.MESH),
            pltpu.make_async_remote_copy(
                src_ref=gx_ref.at[pl.ds(mh, mh), cl],
                dst_ref=gx_ref.at[pl.ds(mh, mh), cl],
                send_sem=sx_l.at[h], recv_sem=rx_l.at[h],
                device_id=(left,), device_id_type=pl.DeviceIdType.MESH),
            pltpu.make_async_remote_copy(
                src_ref=gw_ref.at[cr, pl.ds(0, nh)],
                dst_ref=gw_ref.at[cr, pl.ds(0, nh)],
                send_sem=sw_r.at[h], recv_sem=rw_r.at[h],
                device_id=(right,), device_id_type=pl.DeviceIdType.MESH),
            pltpu.make_async_remote_copy(
                src_ref=gw_ref.at[cl, pl.ds(nh, nh)],
                dst_ref=gw_ref.at[cl, pl.ds(nh, nh)],
                send_sem=sw_l.at[h], recv_sem=rw_l.at[h],
                device_id=(left,), device_id_type=pl.DeviceIdType.MESH),
        ]
        for r in rdmas:
            r.start()
        for r in rdmas:
            r.wait_recv()
        all_rdmas.extend(rdmas)

    for r in all_rdmas:
        r.wait_send()


def _allgather_inputs(x, w_mat):
    m, k_per = x.shape
    _, n = w_mat.shape
    return pl.pallas_call(
        _ag_body,
        out_shape=[
            jax.ShapeDtypeStruct((m, N_DEV * k_per), jnp.int8),
            jax.ShapeDtypeStruct((N_DEV * k_per, n), jnp.int8),
        ],
        in_specs=[
            pl.BlockSpec(memory_space=pltpu.VMEM),
            pl.BlockSpec(memory_space=pltpu.VMEM),
        ],
        out_specs=[
            pl.BlockSpec(memory_space=pltpu.VMEM),
            pl.BlockSpec(memory_space=pltpu.VMEM),
        ],
        scratch_shapes=[pltpu.SemaphoreType.DMA((N_DEV - 1,))] * 8,
        compiler_params=pltpu.CompilerParams(collective_id=0),
    )(x, w_mat)


def _epi_body(y_ref, out_ref):
    y = y_ref[...].astype(jnp.float32)
    out_ref[...] = y * jax.nn.sigmoid(y)


def _epilogue(y):
    m, n = y.shape
    blocks = 32
    bm = m // blocks
    return pl.pallas_call(
        _epi_body,
        out_shape=jax.ShapeDtypeStruct((m, n), jnp.float32),
        grid=(blocks,),
        in_specs=[pl.BlockSpec((bm, n), lambda i: (i, 0))],
        out_specs=pl.BlockSpec((bm, n), lambda i: (i, 0)),
    )(y)


def kernel(x, w_mat, scale_x, scale_w):
    gx, gw = _allgather_inputs(x, w_mat)
    s = (scale_x * scale_w).astype(jnp.float32)[0]
    y = lax.dot_general(
        (gx.astype(jnp.float32) * s).astype(jnp.bfloat16),
        gw.astype(jnp.bfloat16),
        dimension_numbers=(((1,), (0,)), ((), ())),
        preferred_element_type=jnp.float32,
    ).astype(jnp.bfloat16)
    return _epilogue(y)
